# baseline (speedup 1.0000x reference)
import dataclasses
import functools

import jax
import jax.numpy as jnp
from jax import lax
from jax.experimental import pallas as pl
from jax.experimental.pallas import tpu as pltpu
from jax.experimental.pallas import tpu_sc as plsc

ROWS = 8192
CH = 4096
NC = 2
NS = 16
L = 16
NW = NC * NS
RPW = ROWS // NW
RB = 2
NBUF = 4
NCHUNK = RPW // RB
NGROUP = NCHUNK // NBUF
CBLKS = CH // L
CBU = 8


def _permute_sc(z, perm):
  mesh = plsc.VectorSubcoreMesh(core_axis_name="c", subcore_axis_name="s")
  cp = pltpu.CompilerParams()
  if "needs_layout_passes" in pltpu.CompilerParams.__dataclass_fields__:
    cp = dataclasses.replace(cp, needs_layout_passes=False)

  scratch = (
      [pltpu.VMEM((CH,), jnp.int32)]
      + [pltpu.VMEM((RB, CH), jnp.float32) for _ in range(2 * NBUF)]
      + [pltpu.VMEM_SHARED((NS, NBUF, RB, CH), jnp.float32)]
      + [pltpu.SemaphoreType.DMA for _ in range(3 * NBUF)]
  )

  @functools.partial(
      pl.kernel,
      compiler_params=cp,
      out_type=jax.ShapeDtypeStruct((ROWS, CH), jnp.float32),
      mesh=mesh,
      scratch_types=scratch,
  )
  def k(z_hbm, perm_hbm, out_hbm, perm_v, *bufs_and_sems):
    ins = bufs_and_sems[:NBUF]
    outs = bufs_and_sems[NBUF:2 * NBUF]
    shared = bufs_and_sems[2 * NBUF]
    isems = bufs_and_sems[2 * NBUF + 1:3 * NBUF + 1]
    c1sems = bufs_and_sems[3 * NBUF + 1:4 * NBUF + 1]
    c2sems = bufs_and_sems[4 * NBUF + 1:]
    cid = lax.axis_index("c")
    sid = lax.axis_index("s")
    wid = sid * NC + cid
    wbase = wid * RPW

    pltpu.sync_copy(perm_hbm, perm_v)
    for b in range(NBUF):
      pltpu.async_copy(z_hbm.at[pl.ds(wbase + b * RB, RB)], ins[b], isems[b])

    @pl.loop(0, NGROUP)
    def _grp(p):
      for b in range(NBUF):
        kk = p * NBUF + b
        base = wbase + kk * RB
        src = ins[b]
        dst = outs[b]
        pltpu.make_async_copy(z_hbm.at[pl.ds(wbase, RB)], src, isems[b]).wait()

        @pl.when(p > 0)
        def _():
          pltpu.make_async_copy(
              dst, shared.at[sid, b], c1sems[b]).wait()
          pltpu.async_copy(
              shared.at[sid, b],
              out_hbm.at[pl.ds(base - NBUF * RB, RB)], c2sems[b])

        @plsc.parallel_loop(0, CBLKS, step=1, unroll=CBU)
        def _cblk(cb):
          col = cb * L
          idx = perm_v[pl.ds(col, L)]
          for r in range(RB):
            row_idx = jnp.full((L,), r, dtype=jnp.int32)
            dst[r, pl.ds(col, L)] = plsc.load_gather(src, [row_idx, idx])

        @pl.when(p < NGROUP - 1)
        def _():
          pltpu.async_copy(
              z_hbm.at[pl.ds(base + NBUF * RB, RB)], src, isems[b])

        @pl.when(p > 0)
        def _():
          pltpu.make_async_copy(
              shared.at[sid, b],
              out_hbm.at[pl.ds(wbase, RB)], c2sems[b]).wait()

        pltpu.async_copy(dst, shared.at[sid, b], c1sems[b])

    for b in range(NBUF):
      last = wbase + ((NGROUP - 1) * NBUF + b) * RB
      pltpu.make_async_copy(
          outs[b], shared.at[sid, b], c1sems[b]).wait()
      pltpu.async_copy(
          shared.at[sid, b], out_hbm.at[pl.ds(last, RB)], c2sems[b])
      pltpu.make_async_copy(
          shared.at[sid, b], out_hbm.at[pl.ds(wbase, RB)], c2sems[b]).wait()

  return k(z, perm)


def kernel(z, perm):
  z_out = _permute_sc(z, perm.astype(jnp.int32))
  log_det = jnp.zeros((z.shape[0],), dtype=z.dtype)
  return (z_out, log_det)


# speedup vs baseline: 1.0441x; 1.0441x over previous
import dataclasses
import functools

import jax
import jax.numpy as jnp
from jax import lax
from jax.experimental import pallas as pl
from jax.experimental.pallas import tpu as pltpu
from jax.experimental.pallas import tpu_sc as plsc

ROWS = 8192
CH = 4096
NC = 2
NS = 16
L = 16
NW = NC * NS
RPW = ROWS // NW
RB = 2
NBI = 8
NBO = 4
NCHUNK = RPW // RB
NGROUP = NCHUNK // NBI
CBLKS = CH // L
CBU = 8


def _permute_sc(z, perm):
  mesh = plsc.VectorSubcoreMesh(core_axis_name="c", subcore_axis_name="s")
  cp = pltpu.CompilerParams()
  if "needs_layout_passes" in pltpu.CompilerParams.__dataclass_fields__:
    cp = dataclasses.replace(cp, needs_layout_passes=False)

  scratch = (
      [pltpu.VMEM((CH,), jnp.int32)]
      + [pltpu.VMEM((RB, CH), jnp.float32) for _ in range(NBI + NBO)]
      + [pltpu.SemaphoreType.DMA for _ in range(NBI + NBO)]
  )

  @functools.partial(
      pl.kernel,
      compiler_params=cp,
      out_type=jax.ShapeDtypeStruct((ROWS, CH), jnp.float32),
      mesh=mesh,
      scratch_types=scratch,
  )
  def k(z_hbm, perm_hbm, out_hbm, perm_v, *bufs_and_sems):
    ins = bufs_and_sems[:NBI]
    outs = bufs_and_sems[NBI:NBI + NBO]
    isems = bufs_and_sems[NBI + NBO:2 * NBI + NBO]
    osems = bufs_and_sems[2 * NBI + NBO:]
    wid = lax.axis_index("s") * NC + lax.axis_index("c")
    wbase = wid * RPW

    pltpu.sync_copy(perm_hbm, perm_v)
    for b in range(NBI):
      pltpu.async_copy(z_hbm.at[pl.ds(wbase + b * RB, RB)], ins[b], isems[b])

    @pl.loop(0, NGROUP)
    def _grp(p):
      for b in range(NBI):
        kk = p * NBI + b
        o = b % NBO
        base = wbase + kk * RB
        src = ins[b]
        dst = outs[o]
        pltpu.make_async_copy(z_hbm.at[pl.ds(wbase, RB)], src, isems[b]).wait()

        def _wait_out():
          pltpu.make_async_copy(
              dst, out_hbm.at[pl.ds(wbase, RB)], osems[o]).wait()
        if b < NBO:
          pl.when(p > 0)(_wait_out)
        else:
          _wait_out()

        @plsc.parallel_loop(0, CBLKS, step=1, unroll=CBU)
        def _cblk(cb):
          col = cb * L
          idx = perm_v[pl.ds(col, L)]
          for r in range(RB):
            row_idx = jnp.full((L,), r, dtype=jnp.int32)
            dst[r, pl.ds(col, L)] = plsc.load_gather(src, [row_idx, idx])

        @pl.when(p < NGROUP - 1)
        def _():
          pltpu.async_copy(
              z_hbm.at[pl.ds(base + NBI * RB, RB)], src, isems[b])

        pltpu.async_copy(dst, out_hbm.at[pl.ds(base, RB)], osems[o])

    for o in range(NBO):
      pltpu.make_async_copy(
          outs[o], out_hbm.at[pl.ds(wbase, RB)], osems[o]).wait()

  return k(z, perm)


def kernel(z, perm):
  z_out = _permute_sc(z, perm.astype(jnp.int32))
  log_det = jnp.zeros((z.shape[0],), dtype=z.dtype)
  return (z_out, log_det)


# speedup vs baseline: 1.0528x; 1.0083x over previous
import dataclasses
import functools

import jax
import jax.numpy as jnp
from jax import lax
from jax.experimental import pallas as pl
from jax.experimental.pallas import tpu as pltpu
from jax.experimental.pallas import tpu_sc as plsc

ROWS = 8192
CH = 4096
NC = 2
NS = 16
L = 16
NW = NC * NS
RPW = ROWS // NW
RB = 2
NBUF = 4
NCHUNK = RPW // RB
NGROUP = NCHUNK // NBUF
CBLKS = CH // L
CBU = 8


def _permute_sc(z, perm):
  mesh = plsc.VectorSubcoreMesh(core_axis_name="c", subcore_axis_name="s")
  cp = pltpu.CompilerParams()
  if "needs_layout_passes" in pltpu.CompilerParams.__dataclass_fields__:
    cp = dataclasses.replace(cp, needs_layout_passes=False)

  scratch = (
      [pltpu.VMEM((CH,), jnp.int32)]
      + [pltpu.VMEM((RB, CH), jnp.float32) for _ in range(2 * NBUF)]
      + [pltpu.SemaphoreType.DMA for _ in range(2 * NBUF)]
  )

  @functools.partial(
      pl.kernel,
      compiler_params=cp,
      out_type=jax.ShapeDtypeStruct((ROWS, CH), jnp.float32),
      mesh=mesh,
      scratch_types=scratch,
  )
  def k(z_hbm, perm_hbm, out_hbm, perm_v, *bufs_and_sems):
    ins = bufs_and_sems[:NBUF]
    outs = bufs_and_sems[NBUF:2 * NBUF]
    isems = bufs_and_sems[2 * NBUF:3 * NBUF]
    osems = bufs_and_sems[3 * NBUF:]
    wid = lax.axis_index("s") * NC + lax.axis_index("c")
    wbase = wid * RPW

    pltpu.sync_copy(perm_hbm, perm_v)
    for b in range(NBUF):
      pltpu.async_copy(z_hbm.at[pl.ds(wbase + b * RB, RB)], ins[b], isems[b])

    @pl.loop(0, NGROUP)
    def _grp(p):
      for b in range(NBUF):
        kk = p * NBUF + b
        base = wbase + kk * RB
        src = ins[b]
        dst = outs[b]
        pltpu.make_async_copy(z_hbm.at[pl.ds(wbase, RB)], src, isems[b]).wait()
        @pl.when(p > 0)
        def _():
          pltpu.make_async_copy(
              dst, out_hbm.at[pl.ds(wbase, RB)], osems[b]).wait()

        @plsc.parallel_loop(0, CBLKS, step=1, unroll=CBU)
        def _cblk(cb):
          col = cb * L
          idx = perm_v[pl.ds(col, L)]
          for r in range(RB):
            row_idx = jnp.full((L,), r, dtype=jnp.int32)
            dst[r, pl.ds(col, L)] = plsc.load_gather(src, [row_idx, idx])

        @pl.when(p < NGROUP - 1)
        def _():
          pltpu.async_copy(
              z_hbm.at[pl.ds(base + NBUF * RB, RB)], src, isems[b])

        pltpu.async_copy(dst, out_hbm.at[pl.ds(base, RB)], osems[b])

    for b in range(NBUF):
      pltpu.make_async_copy(
          outs[b], out_hbm.at[pl.ds(wbase, RB)], osems[b]).wait()

  return k(z, perm)


def kernel(z, perm):
  z_out = _permute_sc(z, perm.astype(jnp.int32))
  log_det = jnp.zeros((z.shape[0],), dtype=z.dtype)
  return (z_out, log_det)
